# baseline (device time: 36447 ns/iter reference)
import contextlib
import os

import jax
import jax.numpy as jnp
from jax import lax
from jax.experimental import pallas as pl
from jax.experimental.pallas import tpu as pltpu

_PROF = os.environ.get("KERNEL_PROF_SCOPES", "0") == "1"
_ABLATE = os.environ.get("KERNEL_ABLATE", "")
_RDMA = _ABLATE != "compute"


def _scope(name):
    return jax.named_scope(name) if _PROF else contextlib.nullcontext()


N_DEV = 4
B, SQ, SKV_SH, HQ, H_SH, DH = 2, 256, 256, 16, 4, 64
D_MODEL = 512
WINDOW = 128
HD = H_SH * DH
KV1 = 128
SKV = SKV_SH + KV1
QTR = SQ // N_DEV


def kernel(x, Wq, K_ext, V_ext, Wo):
    def body(x_ref, wq_ref, k_ref, v_ref, wo_ref, out_ref,
             kstage, vstage, kbuf0, vbuf0, kbuf1, vbuf1, wstage,
             rs_stage, rs_recv, ag_stage, ag_recv,
             kv_send, kv_recv, rs_send_sem, rs_recv_sem,
             ag_send_sem, ag_recv_sem):
        my_pos = lax.axis_index("i")

        if not _RDMA:
            kbuf0[...] = k_ref[:, :, 0:4, :].astype(jnp.bfloat16).reshape(
                B, SKV_SH, HD)
            vbuf0[...] = v_ref[:, :, 0:4, :].astype(jnp.bfloat16).reshape(
                B, SKV_SH, HD)
            kbuf1[...] = k_ref[:, :KV1, 0:4, :].astype(jnp.bfloat16).reshape(
                B, KV1, HD)
            vbuf1[...] = v_ref[:, :KV1, 0:4, :].astype(jnp.bfloat16).reshape(
                B, KV1, HD)

        def kv_descriptors(src):
            rows = SKV_SH if src == 0 else KV1
            kds, vds = [], []
            targets = [j for j in range(N_DEV) if j != src]
            for ti, j in enumerate(targets):
                for t, (stage, buf0, buf1, out) in enumerate(
                        ((kstage, kbuf0, kbuf1, kds),
                         (vstage, vbuf0, vbuf1, vds))):
                    dst = buf0 if src == 0 else buf1
                    out.append(pltpu.make_async_remote_copy(
                        src_ref=stage.at[:, pl.ds(0, rows), pl.ds(HD * j, HD)],
                        dst_ref=dst,
                        send_sem=kv_send.at[ti, t],
                        recv_sem=kv_recv.at[src, t],
                        device_id=(j,),
                        device_id_type=pl.DeviceIdType.MESH,
                    ))
            return kds, vds

        def kv_wait_recv(t):
            bufs = (kbuf0, kbuf1) if t == 0 else (vbuf0, vbuf1)
            for src in range(2):
                @pl.when(my_pos != src)
                def _(src=src):
                    rows = SKV_SH if src == 0 else KV1
                    pltpu.make_async_remote_copy(
                        src_ref=kstage.at[:, pl.ds(0, rows), pl.ds(0, HD)],
                        dst_ref=bufs[src],
                        send_sem=kv_send.at[0, t],
                        recv_sem=kv_recv.at[src, t],
                        device_id=(src,),
                        device_id_type=pl.DeviceIdType.MESH,
                    ).wait_recv()

        if _RDMA:
            with _scope("barrier"):
                barrier_sem = pltpu.get_barrier_semaphore()
                for p in range(1, N_DEV):
                    pl.semaphore_signal(
                        barrier_sem, inc=1,
                        device_id=((my_pos + p) % N_DEV,),
                        device_id_type=pl.DeviceIdType.MESH,
                    )
                pl.semaphore_wait(barrier_sem, N_DEV - 1)

            with _scope("kv_stage_send"):
                for src in range(2):
                    @pl.when(my_pos == src)
                    def _(src=src):
                        rows = SKV_SH if src == 0 else KV1
                        kstage[:, :rows, :] = k_ref[:, :rows, :, :].astype(
                            jnp.bfloat16).reshape(B, rows, HQ * DH)
                        vstage[:, :rows, :] = v_ref[:, :rows, :, :].astype(
                            jnp.bfloat16).reshape(B, rows, HQ * DH)
                        kds, vds = kv_descriptors(src)
                        for d in kds + vds:
                            d.start()
                        dst_k = kbuf0 if src == 0 else kbuf1
                        dst_v = vbuf0 if src == 0 else vbuf1
                        dst_k[...] = kstage[:, :rows, HD * src:HD * (src + 1)]
                        dst_v[...] = vstage[:, :rows, HD * src:HD * (src + 1)]

        with _scope("qproj"):
            wq_b = (wq_ref[...] * 0.125).astype(jnp.bfloat16)
            wo_b = wo_ref[...].astype(jnp.bfloat16)
            q_all = []
            for b in range(B):
                xb = x_ref[b].astype(jnp.bfloat16)
                q_all.append(jnp.dot(xb, wq_b,
                                     preferred_element_type=jnp.float32))

        if _RDMA:
            with _scope("k_wait_recv"):
                kv_wait_recv(0)

        attn_scope = _scope("attn_scores")
        attn_scope.__enter__()
        qi = lax.broadcasted_iota(jnp.int32, (SQ, SKV), 0)
        kj = lax.broadcasted_iota(jnp.int32, (SQ, SKV), 1)
        mask = jnp.abs(qi - kj) <= WINDOW

        for b in range(B):
            k_cat = jnp.concatenate([kbuf0[b], kbuf1[b]], axis=0)
            for h in range(H_SH):
                q_h = q_all[b][:, DH * h:DH * (h + 1)].astype(jnp.bfloat16)
                k_h = k_cat[:, DH * h:DH * (h + 1)]
                s = lax.dot_general(
                    q_h, k_h, (((1,), (1,)), ((), ())),
                    preferred_element_type=jnp.float32)
                s = jnp.where(mask, s, -1e9)
                m = jnp.max(s, axis=1, keepdims=True)
                w = jnp.exp(s - m)
                wstage[b, h] = (w / jnp.sum(w, axis=1, keepdims=True)).astype(
                    jnp.bfloat16)
        attn_scope.__exit__(None, None, None)

        if _RDMA:
            with _scope("v_wait_recv"):
                kv_wait_recv(1)

        rs_rdmas = []
        if _RDMA:
            for p in range(1, N_DEV):
                tgt = (my_pos + p) % N_DEV
                rs_rdmas.append(pltpu.make_async_remote_copy(
                    src_ref=rs_stage.at[:, pl.ds(tgt * QTR, QTR), :],
                    dst_ref=rs_recv.at[p - 1],
                    send_sem=rs_send_sem.at[p - 1],
                    recv_sem=rs_recv_sem.at[p - 1],
                    device_id=(tgt,),
                    device_id_type=pl.DeviceIdType.MESH,
                ))

        attn_scope = _scope("attn_ctx")
        attn_scope.__enter__()
        v_cats = [jnp.concatenate([vbuf0[b], vbuf1[b]], axis=0)
                  for b in range(B)]
        for p in range(1, N_DEV + 1):
            qb = (my_pos + p) % N_DEV
            row0 = qb * QTR
            for b in range(B):
                ctx_cols = []
                for h in range(H_SH):
                    w_blk = wstage[b, h, pl.ds(row0, QTR), :]
                    ctx_cols.append(jnp.dot(
                        w_blk, v_cats[b][:, DH * h:DH * (h + 1)],
                        preferred_element_type=jnp.float32))
                ctx_blk = jnp.concatenate(ctx_cols, axis=1).astype(
                    jnp.bfloat16)
                part_blk = jnp.dot(ctx_blk, wo_b,
                                   preferred_element_type=jnp.float32)
                rs_stage[b, pl.ds(row0, QTR), :] = part_blk.astype(
                    jnp.bfloat16)
                out_ref[b, pl.ds(row0, QTR), :] = part_blk
            if _RDMA and p <= N_DEV - 1:
                rs_rdmas[p - 1].start()
        attn_scope.__exit__(None, None, None)

        if _RDMA:
            with _scope("kv_wait_send"):
                for src in range(2):
                    @pl.when(my_pos == src)
                    def _(src=src):
                        kds, vds = kv_descriptors(src)
                        for d in kds + vds:
                            d.wait_send()

            with _scope("rs_wait_recv"):
                for r in rs_rdmas:
                    r.wait_recv()

            with _scope("reduce"):
                for b in range(B):
                    red_b = out_ref[b, pl.ds(my_pos * QTR, QTR), :]
                    for p in range(1, N_DEV):
                        red_b = red_b + rs_recv[p - 1, b].astype(jnp.float32)
                    out_ref[b, pl.ds(my_pos * QTR, QTR), :] = red_b
                    ag_stage[b] = red_b.astype(jnp.bfloat16)

            with _scope("ag_start"):
                ag_rdmas = []
                for p in range(1, N_DEV):
                    tgt = (my_pos + p) % N_DEV
                    ag_rdmas.append(pltpu.make_async_remote_copy(
                        src_ref=ag_stage,
                        dst_ref=ag_recv.at[p - 1],
                        send_sem=ag_send_sem.at[p - 1],
                        recv_sem=ag_recv_sem.at[p - 1],
                        device_id=(tgt,),
                        device_id_type=pl.DeviceIdType.MESH,
                    ))
                for r in ag_rdmas:
                    r.start()
            with _scope("ag_wait_recv"):
                for r in ag_rdmas:
                    r.wait_recv()
            with _scope("assemble"):
                for p in range(1, N_DEV):
                    src = (my_pos - p) % N_DEV
                    for b in range(B):
                        out_ref[b, pl.ds(src * QTR, QTR), :] = (
                            ag_recv[p - 1, b].astype(jnp.float32))

            with _scope("tail_wait_send"):
                for r in rs_rdmas:
                    r.wait_send()
                for r in ag_rdmas:
                    r.wait_send()

    return pl.pallas_call(
        body,
        out_shape=jax.ShapeDtypeStruct((B, SQ, D_MODEL), jnp.float32),
        in_specs=[pl.BlockSpec(memory_space=pltpu.VMEM)] * 5,
        out_specs=pl.BlockSpec(memory_space=pltpu.VMEM),
        scratch_shapes=[
            pltpu.VMEM((B, SKV_SH, HQ * DH), jnp.bfloat16),
            pltpu.VMEM((B, SKV_SH, HQ * DH), jnp.bfloat16),
            pltpu.VMEM((B, SKV_SH, HD), jnp.bfloat16),
            pltpu.VMEM((B, SKV_SH, HD), jnp.bfloat16),
            pltpu.VMEM((B, KV1, HD), jnp.bfloat16),
            pltpu.VMEM((B, KV1, HD), jnp.bfloat16),
            pltpu.VMEM((B, H_SH, SQ, SKV), jnp.bfloat16),
            pltpu.VMEM((B, SQ, D_MODEL), jnp.bfloat16),
            pltpu.VMEM((N_DEV - 1, B, QTR, D_MODEL), jnp.bfloat16),
            pltpu.VMEM((B, QTR, D_MODEL), jnp.bfloat16),
            pltpu.VMEM((N_DEV - 1, B, QTR, D_MODEL), jnp.bfloat16),
            pltpu.SemaphoreType.DMA((N_DEV - 1, 2)),
            pltpu.SemaphoreType.DMA((2, 2)),
            pltpu.SemaphoreType.DMA((N_DEV - 1,)),
            pltpu.SemaphoreType.DMA((N_DEV - 1,)),
            pltpu.SemaphoreType.DMA((N_DEV - 1,)),
            pltpu.SemaphoreType.DMA((N_DEV - 1,)),
        ],
        compiler_params=(pltpu.CompilerParams(collective_id=0) if _RDMA
                         else pltpu.CompilerParams()),
    )(x, Wq, K_ext, V_ext, Wo)


# device time: 34480 ns/iter; 1.0570x vs baseline; 1.0570x over previous
import contextlib
import os

import jax
import jax.numpy as jnp
from jax import lax
from jax.experimental import pallas as pl
from jax.experimental.pallas import tpu as pltpu

_PROF = os.environ.get("KERNEL_PROF_SCOPES", "0") == "1"
_ABLATE = os.environ.get("KERNEL_ABLATE", "")
_RDMA = _ABLATE != "compute"


def _scope(name):
    return jax.named_scope(name) if _PROF else contextlib.nullcontext()


N_DEV = 4
B, SQ, SKV_SH, HQ, H_SH, DH = 2, 256, 256, 16, 4, 64
D_MODEL = 512
WINDOW = 128
HD = H_SH * DH
KV1 = 128
SKV = SKV_SH + KV1
QTR = SQ // N_DEV


def kernel(x, Wq, K_ext, V_ext, Wo):
    def body(x_ref, wq_ref, k_ref, v_ref, wo_ref, out_ref,
             kstage, vstage, kbuf0, vbuf0, kbuf1, vbuf1,
             rs_stage, rs_recv, ag_stage, ag_recv,
             kv_send, kv_recv, rs_send_sem, rs_recv_sem,
             ag_send_sem, ag_recv_sem):
        my_pos = lax.axis_index("i")

        if not _RDMA:
            kbuf0[...] = k_ref[:, :, 0:4, :].astype(jnp.bfloat16).reshape(
                B, SKV_SH, HD)
            vbuf0[...] = v_ref[:, :, 0:4, :].astype(jnp.bfloat16).reshape(
                B, SKV_SH, HD)
            kbuf1[...] = k_ref[:, :KV1, 0:4, :].astype(jnp.bfloat16).reshape(
                B, KV1, HD)
            vbuf1[...] = v_ref[:, :KV1, 0:4, :].astype(jnp.bfloat16).reshape(
                B, KV1, HD)

        def kv_descriptors(src):
            rows = SKV_SH if src == 0 else KV1
            kds, vds = [], []
            targets = [j for j in range(N_DEV) if j != src]
            for ti, j in enumerate(targets):
                for t, (stage, buf0, buf1, out) in enumerate(
                        ((kstage, kbuf0, kbuf1, kds),
                         (vstage, vbuf0, vbuf1, vds))):
                    dst = buf0 if src == 0 else buf1
                    out.append(pltpu.make_async_remote_copy(
                        src_ref=stage.at[:, pl.ds(0, rows), pl.ds(HD * j, HD)],
                        dst_ref=dst,
                        send_sem=kv_send.at[ti, t],
                        recv_sem=kv_recv.at[src, t],
                        device_id=(j,),
                        device_id_type=pl.DeviceIdType.MESH,
                    ))
            return kds, vds

        def kv_wait_recv(t):
            bufs = (kbuf0, kbuf1) if t == 0 else (vbuf0, vbuf1)
            for src in range(2):
                @pl.when(my_pos != src)
                def _(src=src):
                    rows = SKV_SH if src == 0 else KV1
                    pltpu.make_async_remote_copy(
                        src_ref=kstage.at[:, pl.ds(0, rows), pl.ds(0, HD)],
                        dst_ref=bufs[src],
                        send_sem=kv_send.at[0, t],
                        recv_sem=kv_recv.at[src, t],
                        device_id=(src,),
                        device_id_type=pl.DeviceIdType.MESH,
                    ).wait_recv()

        if _RDMA:
            with _scope("barrier"):
                barrier_sem = pltpu.get_barrier_semaphore()
                for p in range(1, N_DEV):
                    pl.semaphore_signal(
                        barrier_sem, inc=1,
                        device_id=((my_pos + p) % N_DEV,),
                        device_id_type=pl.DeviceIdType.MESH,
                    )
                pl.semaphore_wait(barrier_sem, N_DEV - 1)

            with _scope("kv_stage_send"):
                for src in range(2):
                    @pl.when(my_pos == src)
                    def _(src=src):
                        rows = SKV_SH if src == 0 else KV1
                        kds, vds = kv_descriptors(src)
                        kstage[:, :rows, :] = k_ref[:, :rows, :, :].astype(
                            jnp.bfloat16).reshape(B, rows, HQ * DH)
                        for d in kds:
                            d.start()
                        vstage[:, :rows, :] = v_ref[:, :rows, :, :].astype(
                            jnp.bfloat16).reshape(B, rows, HQ * DH)
                        for d in vds:
                            d.start()
                        dst_k = kbuf0 if src == 0 else kbuf1
                        dst_v = vbuf0 if src == 0 else vbuf1
                        dst_k[...] = kstage[:, :rows, HD * src:HD * (src + 1)]
                        dst_v[...] = vstage[:, :rows, HD * src:HD * (src + 1)]

        with _scope("qproj"):
            wq_b = (wq_ref[...] * 0.125).astype(jnp.bfloat16)
            wo_b = wo_ref[...].astype(jnp.bfloat16)
            q_all = []
            for b in range(B):
                xb = x_ref[b].astype(jnp.bfloat16)
                q_all.append(jnp.dot(xb, wq_b,
                                     preferred_element_type=jnp.float32))

        if _RDMA:
            with _scope("k_wait_recv"):
                kv_wait_recv(0)

        attn_scope = _scope("attn_scores")
        attn_scope.__enter__()
        qi = lax.broadcasted_iota(jnp.int32, (SQ, SKV), 0)
        kj = lax.broadcasted_iota(jnp.int32, (SQ, SKV), 1)
        mask = jnp.abs(qi - kj) <= WINDOW

        ws = []
        for b in range(B):
            k_cat = jnp.concatenate([kbuf0[b], kbuf1[b]], axis=0)
            for h in range(H_SH):
                q_h = q_all[b][:, DH * h:DH * (h + 1)].astype(jnp.bfloat16)
                k_h = k_cat[:, DH * h:DH * (h + 1)]
                s = lax.dot_general(
                    q_h, k_h, (((1,), (1,)), ((), ())),
                    preferred_element_type=jnp.float32)
                s = jnp.where(mask, s, -1e9)
                m = jnp.max(s, axis=1, keepdims=True)
                w = jnp.exp(s - m)
                ws.append((w / jnp.sum(w, axis=1, keepdims=True)).astype(
                    jnp.bfloat16))
        attn_scope.__exit__(None, None, None)

        if _RDMA:
            with _scope("v_wait_recv"):
                kv_wait_recv(1)

        def rs_descriptors(b):
            ds = []
            for p in range(1, N_DEV):
                tgt = (my_pos + p) % N_DEV
                ds.append(pltpu.make_async_remote_copy(
                    src_ref=rs_stage.at[b, pl.ds(tgt * QTR, QTR), :],
                    dst_ref=rs_recv.at[p - 1, b],
                    send_sem=rs_send_sem.at[p - 1, b],
                    recv_sem=rs_recv_sem.at[p - 1, b],
                    device_id=(tgt,),
                    device_id_type=pl.DeviceIdType.MESH,
                ))
            return ds

        attn_scope = _scope("attn_ctx")
        attn_scope.__enter__()
        rs_rdmas = []
        for b in range(B):
            v_cat = jnp.concatenate([vbuf0[b], vbuf1[b]], axis=0)
            ctx_cols = []
            for h in range(H_SH):
                ctx_cols.append(jnp.dot(
                    ws[b * H_SH + h], v_cat[:, DH * h:DH * (h + 1)],
                    preferred_element_type=jnp.float32))
            ctx_b = jnp.concatenate(ctx_cols, axis=1).astype(jnp.bfloat16)
            part_b = jnp.dot(ctx_b, wo_b,
                             preferred_element_type=jnp.float32)
            rs_stage[b] = part_b.astype(jnp.bfloat16)
            out_ref[b] = part_b
            if _RDMA:
                ds = rs_descriptors(b)
                for r in ds:
                    r.start()
                rs_rdmas.extend(ds)
        attn_scope.__exit__(None, None, None)

        if _RDMA:
            with _scope("kv_wait_send"):
                for src in range(2):
                    @pl.when(my_pos == src)
                    def _(src=src):
                        kds, vds = kv_descriptors(src)
                        for d in kds + vds:
                            d.wait_send()

            with _scope("rs_wait_recv"):
                for r in rs_rdmas:
                    r.wait_recv()

            with _scope("reduce"):
                for b in range(B):
                    red_b = out_ref[b, pl.ds(my_pos * QTR, QTR), :]
                    for p in range(1, N_DEV):
                        red_b = red_b + rs_recv[p - 1, b].astype(jnp.float32)
                    out_ref[b, pl.ds(my_pos * QTR, QTR), :] = red_b
                    ag_stage[b] = red_b.astype(jnp.bfloat16)

            with _scope("ag_start"):
                ag_rdmas = []
                for p in range(1, N_DEV):
                    tgt = (my_pos + p) % N_DEV
                    ag_rdmas.append(pltpu.make_async_remote_copy(
                        src_ref=ag_stage,
                        dst_ref=ag_recv.at[p - 1],
                        send_sem=ag_send_sem.at[p - 1],
                        recv_sem=ag_recv_sem.at[p - 1],
                        device_id=(tgt,),
                        device_id_type=pl.DeviceIdType.MESH,
                    ))
                for r in ag_rdmas:
                    r.start()
            with _scope("ag_wait_recv"):
                for r in ag_rdmas:
                    r.wait_recv()
            with _scope("assemble"):
                for p in range(1, N_DEV):
                    src = (my_pos - p) % N_DEV
                    for b in range(B):
                        out_ref[b, pl.ds(src * QTR, QTR), :] = (
                            ag_recv[p - 1, b].astype(jnp.float32))

            with _scope("tail_wait_send"):
                for r in rs_rdmas:
                    r.wait_send()
                for r in ag_rdmas:
                    r.wait_send()

    return pl.pallas_call(
        body,
        out_shape=jax.ShapeDtypeStruct((B, SQ, D_MODEL), jnp.float32),
        in_specs=[pl.BlockSpec(memory_space=pltpu.VMEM)] * 5,
        out_specs=pl.BlockSpec(memory_space=pltpu.VMEM),
        scratch_shapes=[
            pltpu.VMEM((B, SKV_SH, HQ * DH), jnp.bfloat16),
            pltpu.VMEM((B, SKV_SH, HQ * DH), jnp.bfloat16),
            pltpu.VMEM((B, SKV_SH, HD), jnp.bfloat16),
            pltpu.VMEM((B, SKV_SH, HD), jnp.bfloat16),
            pltpu.VMEM((B, KV1, HD), jnp.bfloat16),
            pltpu.VMEM((B, KV1, HD), jnp.bfloat16),
            pltpu.VMEM((B, SQ, D_MODEL), jnp.bfloat16),
            pltpu.VMEM((N_DEV - 1, B, QTR, D_MODEL), jnp.bfloat16),
            pltpu.VMEM((B, QTR, D_MODEL), jnp.bfloat16),
            pltpu.VMEM((N_DEV - 1, B, QTR, D_MODEL), jnp.bfloat16),
            pltpu.SemaphoreType.DMA((N_DEV - 1, 2)),
            pltpu.SemaphoreType.DMA((2, 2)),
            pltpu.SemaphoreType.DMA((N_DEV - 1, B)),
            pltpu.SemaphoreType.DMA((N_DEV - 1, B)),
            pltpu.SemaphoreType.DMA((N_DEV - 1,)),
            pltpu.SemaphoreType.DMA((N_DEV - 1,)),
        ],
        compiler_params=(pltpu.CompilerParams(collective_id=0) if _RDMA
                         else pltpu.CompilerParams()),
    )(x, Wq, K_ext, V_ext, Wo)


# device time: 33805 ns/iter; 1.0782x vs baseline; 1.0200x over previous
import contextlib
import os

import jax
import jax.numpy as jnp
from jax import lax
from jax.experimental import pallas as pl
from jax.experimental.pallas import tpu as pltpu

_PROF = os.environ.get("KERNEL_PROF_SCOPES", "0") == "1"
_ABLATE = os.environ.get("KERNEL_ABLATE", "")
_RDMA = _ABLATE != "compute"


def _scope(name):
    return jax.named_scope(name) if _PROF else contextlib.nullcontext()


N_DEV = 4
B, SQ, SKV_SH, HQ, H_SH, DH = 2, 256, 256, 16, 4, 64
D_MODEL = 512
WINDOW = 128
HD = H_SH * DH
KV1 = 128
SKV = SKV_SH + KV1
QTR = SQ // N_DEV


def kernel(x, Wq, K_ext, V_ext, Wo):
    def body(x_ref, wq_ref, k_ref, v_ref, wo_ref, out_ref,
             kstage, vstage, kbuf0, vbuf0, kbuf1, vbuf1,
             rs_stage, rs_recv, ag_stage, ag_recv,
             kv_send, kv_recv, rs_send_sem, rs_recv_sem,
             ag_send_sem, ag_recv_sem):
        my_pos = lax.axis_index("i")

        if not _RDMA:
            kbuf0[...] = k_ref[:, :, 0:4, :].astype(jnp.bfloat16).reshape(
                B, SKV_SH, HD)
            vbuf0[...] = v_ref[:, :, 0:4, :].astype(jnp.bfloat16).reshape(
                B, SKV_SH, HD)
            kbuf1[...] = k_ref[:, :KV1, 0:4, :].astype(jnp.bfloat16).reshape(
                B, KV1, HD)
            vbuf1[...] = v_ref[:, :KV1, 0:4, :].astype(jnp.bfloat16).reshape(
                B, KV1, HD)

        def kv_descriptors(src):
            rows = SKV_SH if src == 0 else KV1
            kds, vds = [], []
            targets = [j for j in range(N_DEV) if j != src]
            for ti, j in enumerate(targets):
                for t, (stage, buf0, buf1, out) in enumerate(
                        ((kstage, kbuf0, kbuf1, kds),
                         (vstage, vbuf0, vbuf1, vds))):
                    dst = buf0 if src == 0 else buf1
                    out.append(pltpu.make_async_remote_copy(
                        src_ref=stage.at[:, pl.ds(0, rows), pl.ds(HD * j, HD)],
                        dst_ref=dst,
                        send_sem=kv_send.at[ti, t],
                        recv_sem=kv_recv.at[src, t],
                        device_id=(j,),
                        device_id_type=pl.DeviceIdType.MESH,
                    ))
            return kds, vds

        def kv_wait_recv(t):
            bufs = (kbuf0, kbuf1) if t == 0 else (vbuf0, vbuf1)
            for src in range(2):
                @pl.when(my_pos != src)
                def _(src=src):
                    rows = SKV_SH if src == 0 else KV1
                    pltpu.make_async_remote_copy(
                        src_ref=kstage.at[:, pl.ds(0, rows), pl.ds(0, HD)],
                        dst_ref=bufs[src],
                        send_sem=kv_send.at[0, t],
                        recv_sem=kv_recv.at[src, t],
                        device_id=(src,),
                        device_id_type=pl.DeviceIdType.MESH,
                    ).wait_recv()

        if _RDMA:
            with _scope("barrier"):
                barrier_sem = pltpu.get_barrier_semaphore()
                for p in range(1, N_DEV):
                    pl.semaphore_signal(
                        barrier_sem, inc=1,
                        device_id=((my_pos + p) % N_DEV,),
                        device_id_type=pl.DeviceIdType.MESH,
                    )
                pl.semaphore_wait(barrier_sem, N_DEV - 1)

            with _scope("kv_stage_send"):
                for src in range(2):
                    @pl.when(my_pos == src)
                    def _(src=src):
                        rows = SKV_SH if src == 0 else KV1
                        kds, vds = kv_descriptors(src)
                        kstage[:, :rows, :] = k_ref[:, :rows, :, :].astype(
                            jnp.bfloat16).reshape(B, rows, HQ * DH)
                        for d in kds:
                            d.start()
                        vstage[:, :rows, :] = v_ref[:, :rows, :, :].astype(
                            jnp.bfloat16).reshape(B, rows, HQ * DH)
                        for d in vds:
                            d.start()
                        dst_k = kbuf0 if src == 0 else kbuf1
                        dst_v = vbuf0 if src == 0 else vbuf1
                        dst_k[...] = kstage[:, :rows, HD * src:HD * (src + 1)]
                        dst_v[...] = vstage[:, :rows, HD * src:HD * (src + 1)]

        with _scope("qproj"):
            wq_b = (wq_ref[...] * 0.125).astype(jnp.bfloat16)
            wo_b = wo_ref[...].astype(jnp.bfloat16)
            q_all = []
            for b in range(B):
                xb = x_ref[b].astype(jnp.bfloat16)
                q_all.append(jnp.dot(xb, wq_b,
                                     preferred_element_type=jnp.float32))

        if _RDMA:
            with _scope("k_wait_recv"):
                kv_wait_recv(0)

        attn_scope = _scope("attn_scores")
        attn_scope.__enter__()
        qi = lax.broadcasted_iota(jnp.int32, (SQ, SKV), 0)
        kj = lax.broadcasted_iota(jnp.int32, (SQ, SKV), 1)
        bias = jnp.where(jnp.abs(qi - kj) <= WINDOW, 0.0, -1e9)

        ws, sums = [], []
        for b in range(B):
            k_cat = jnp.concatenate([kbuf0[b], kbuf1[b]], axis=0)
            for h in range(H_SH):
                q_h = q_all[b][:, DH * h:DH * (h + 1)].astype(jnp.bfloat16)
                k_h = k_cat[:, DH * h:DH * (h + 1)]
                s = lax.dot_general(
                    q_h, k_h, (((1,), (1,)), ((), ())),
                    preferred_element_type=jnp.float32)
                w = jnp.exp(s + bias)
                sums.append(jnp.sum(w, axis=1, keepdims=True))
                ws.append(w.astype(jnp.bfloat16))
        attn_scope.__exit__(None, None, None)

        if _RDMA:
            with _scope("v_wait_recv"):
                kv_wait_recv(1)

        def rs_descriptors(b):
            ds = []
            for p in range(1, N_DEV):
                tgt = (my_pos + p) % N_DEV
                ds.append(pltpu.make_async_remote_copy(
                    src_ref=rs_stage.at[b, pl.ds(tgt * QTR, QTR), :],
                    dst_ref=rs_recv.at[p - 1, b],
                    send_sem=rs_send_sem.at[p - 1, b],
                    recv_sem=rs_recv_sem.at[p - 1, b],
                    device_id=(tgt,),
                    device_id_type=pl.DeviceIdType.MESH,
                ))
            return ds

        attn_scope = _scope("attn_ctx")
        attn_scope.__enter__()
        rs_rdmas = []
        for b in range(B):
            v_cat = jnp.concatenate([vbuf0[b], vbuf1[b]], axis=0)
            ctx_cols = []
            for h in range(H_SH):
                u = jnp.dot(ws[b * H_SH + h], v_cat[:, DH * h:DH * (h + 1)],
                            preferred_element_type=jnp.float32)
                ctx_cols.append(u / sums[b * H_SH + h])
            ctx_b = jnp.concatenate(ctx_cols, axis=1).astype(jnp.bfloat16)
            part_b = jnp.dot(ctx_b, wo_b,
                             preferred_element_type=jnp.float32)
            rs_stage[b] = part_b.astype(jnp.bfloat16)
            out_ref[b] = part_b
            if _RDMA:
                ds = rs_descriptors(b)
                for r in ds:
                    r.start()
                rs_rdmas.extend(ds)
        attn_scope.__exit__(None, None, None)

        if _RDMA:
            with _scope("kv_wait_send"):
                for src in range(2):
                    @pl.when(my_pos == src)
                    def _(src=src):
                        kds, vds = kv_descriptors(src)
                        for d in kds + vds:
                            d.wait_send()

            ag_rdmas = []
            with _scope("reduce_ag"):
                for b in range(B):
                    for r in rs_rdmas[3 * b:3 * (b + 1)]:
                        r.wait_recv()
                    red_b = out_ref[b, pl.ds(my_pos * QTR, QTR), :]
                    for p in range(1, N_DEV):
                        red_b = red_b + rs_recv[p - 1, b].astype(jnp.float32)
                    out_ref[b, pl.ds(my_pos * QTR, QTR), :] = red_b
                    ag_stage[b] = red_b.astype(jnp.bfloat16)
                    for p in range(1, N_DEV):
                        tgt = (my_pos + p) % N_DEV
                        r = pltpu.make_async_remote_copy(
                            src_ref=ag_stage.at[b],
                            dst_ref=ag_recv.at[p - 1, b],
                            send_sem=ag_send_sem.at[p - 1, b],
                            recv_sem=ag_recv_sem.at[p - 1, b],
                            device_id=(tgt,),
                            device_id_type=pl.DeviceIdType.MESH,
                        )
                        r.start()
                        ag_rdmas.append(r)
            with _scope("ag_wait_recv"):
                for r in ag_rdmas:
                    r.wait_recv()
            with _scope("assemble"):
                for p in range(1, N_DEV):
                    src = (my_pos - p) % N_DEV
                    for b in range(B):
                        out_ref[b, pl.ds(src * QTR, QTR), :] = (
                            ag_recv[p - 1, b].astype(jnp.float32))

            with _scope("tail_wait_send"):
                for r in rs_rdmas:
                    r.wait_send()
                for r in ag_rdmas:
                    r.wait_send()

    return pl.pallas_call(
        body,
        out_shape=jax.ShapeDtypeStruct((B, SQ, D_MODEL), jnp.float32),
        in_specs=[pl.BlockSpec(memory_space=pltpu.VMEM)] * 5,
        out_specs=pl.BlockSpec(memory_space=pltpu.VMEM),
        scratch_shapes=[
            pltpu.VMEM((B, SKV_SH, HQ * DH), jnp.bfloat16),
            pltpu.VMEM((B, SKV_SH, HQ * DH), jnp.bfloat16),
            pltpu.VMEM((B, SKV_SH, HD), jnp.bfloat16),
            pltpu.VMEM((B, SKV_SH, HD), jnp.bfloat16),
            pltpu.VMEM((B, KV1, HD), jnp.bfloat16),
            pltpu.VMEM((B, KV1, HD), jnp.bfloat16),
            pltpu.VMEM((B, SQ, D_MODEL), jnp.bfloat16),
            pltpu.VMEM((N_DEV - 1, B, QTR, D_MODEL), jnp.bfloat16),
            pltpu.VMEM((B, QTR, D_MODEL), jnp.bfloat16),
            pltpu.VMEM((N_DEV - 1, B, QTR, D_MODEL), jnp.bfloat16),
            pltpu.SemaphoreType.DMA((N_DEV - 1, 2)),
            pltpu.SemaphoreType.DMA((2, 2)),
            pltpu.SemaphoreType.DMA((N_DEV - 1, B)),
            pltpu.SemaphoreType.DMA((N_DEV - 1, B)),
            pltpu.SemaphoreType.DMA((N_DEV - 1, B)),
            pltpu.SemaphoreType.DMA((N_DEV - 1, B)),
        ],
        compiler_params=(pltpu.CompilerParams(collective_id=0) if _RDMA
                         else pltpu.CompilerParams()),
    )(x, Wq, K_ext, V_ext, Wo)


# device time: 33586 ns/iter; 1.0852x vs baseline; 1.0065x over previous
import contextlib
import os

import jax
import jax.numpy as jnp
from jax import lax
from jax.experimental import pallas as pl
from jax.experimental.pallas import tpu as pltpu

_PROF = os.environ.get("KERNEL_PROF_SCOPES", "0") == "1"
_ABLATE = os.environ.get("KERNEL_ABLATE", "")
_RDMA = _ABLATE != "compute"
_KV = _ABLATE not in ("compute", "nokv")


def _scope(name):
    return jax.named_scope(name) if _PROF else contextlib.nullcontext()


N_DEV = 4
B, SQ, SKV_SH, HQ, H_SH, DH = 2, 256, 256, 16, 4, 64
D_MODEL = 512
WINDOW = 128
HD = H_SH * DH
KV1 = 128
SKV = SKV_SH + KV1
QTR = SQ // N_DEV


def kernel(x, Wq, K_ext, V_ext, Wo):
    def body(x_ref, wq_ref, k_ref, v_ref, wo_ref, out_ref,
             kstage, vstage, kbuf0, vbuf0, kbuf1, vbuf1,
             rs_stage, rs_recv, ag_stage, ag_recv,
             kv_send, kv_recv, rs_send_sem, rs_recv_sem,
             ag_send_sem, ag_recv_sem):
        my_pos = lax.axis_index("i")

        if not _KV:
            kbuf0[...] = k_ref[:, :, 0:4, :].astype(jnp.bfloat16).reshape(
                B, SKV_SH, HD)
            vbuf0[...] = v_ref[:, :, 0:4, :].astype(jnp.bfloat16).reshape(
                B, SKV_SH, HD)
            kbuf1[...] = k_ref[:, :KV1, 0:4, :].astype(jnp.bfloat16).reshape(
                B, KV1, HD)
            vbuf1[...] = v_ref[:, :KV1, 0:4, :].astype(jnp.bfloat16).reshape(
                B, KV1, HD)

        def kv_descriptors(src):
            rows = SKV_SH if src == 0 else KV1
            kds, vds = [], []
            targets = [j for j in range(N_DEV) if j != src]
            for ti, j in enumerate(targets):
                for t, (stage, buf0, buf1, out) in enumerate(
                        ((kstage, kbuf0, kbuf1, kds),
                         (vstage, vbuf0, vbuf1, vds))):
                    dst = buf0 if src == 0 else buf1
                    out.append(pltpu.make_async_remote_copy(
                        src_ref=stage.at[:, pl.ds(0, rows), pl.ds(HD * j, HD)],
                        dst_ref=dst,
                        send_sem=kv_send.at[ti, t],
                        recv_sem=kv_recv.at[src, t],
                        device_id=(j,),
                        device_id_type=pl.DeviceIdType.MESH,
                    ))
            return kds, vds

        def kv_wait_one(t, src):
            if not _KV:
                return
            bufs = (kbuf0, kbuf1) if t == 0 else (vbuf0, vbuf1)

            @pl.when(my_pos != src)
            def _():
                rows = SKV_SH if src == 0 else KV1
                pltpu.make_async_remote_copy(
                    src_ref=kstage.at[:, pl.ds(0, rows), pl.ds(0, HD)],
                    dst_ref=bufs[src],
                    send_sem=kv_send.at[0, t],
                    recv_sem=kv_recv.at[src, t],
                    device_id=(src,),
                    device_id_type=pl.DeviceIdType.MESH,
                ).wait_recv()

        if _RDMA:
            with _scope("barrier"):
                barrier_sem = pltpu.get_barrier_semaphore()
                for p in range(1, N_DEV):
                    pl.semaphore_signal(
                        barrier_sem, inc=1,
                        device_id=((my_pos + p) % N_DEV,),
                        device_id_type=pl.DeviceIdType.MESH,
                    )
                pl.semaphore_wait(barrier_sem, N_DEV - 1)

        if _KV:
            with _scope("kv_stage_send"):
                for src in range(2):
                    @pl.when(my_pos == src)
                    def _(src=src):
                        rows = SKV_SH if src == 0 else KV1
                        kds, vds = kv_descriptors(src)
                        targets = [j for j in range(N_DEV) if j != src]
                        for ti, j in enumerate(targets):
                            kstage[:, :rows, HD * j:HD * (j + 1)] = (
                                k_ref[:, :rows, H_SH * j:H_SH * (j + 1), :]
                                .astype(jnp.bfloat16).reshape(B, rows, HD))
                            kds[ti].start()
                        for ti, j in enumerate(targets):
                            vstage[:, :rows, HD * j:HD * (j + 1)] = (
                                v_ref[:, :rows, H_SH * j:H_SH * (j + 1), :]
                                .astype(jnp.bfloat16).reshape(B, rows, HD))
                            vds[ti].start()
                        dst_k = kbuf0 if src == 0 else kbuf1
                        dst_v = vbuf0 if src == 0 else vbuf1
                        dst_k[...] = (
                            k_ref[:, :rows, H_SH * src:H_SH * (src + 1), :]
                            .astype(jnp.bfloat16).reshape(B, rows, HD))
                        dst_v[...] = (
                            v_ref[:, :rows, H_SH * src:H_SH * (src + 1), :]
                            .astype(jnp.bfloat16).reshape(B, rows, HD))

        with _scope("qproj"):
            wq_b = (wq_ref[...] * 0.125).astype(jnp.bfloat16)
            wo_b = wo_ref[...].astype(jnp.bfloat16)
            q_all = []
            for b in range(B):
                xb = x_ref[b].astype(jnp.bfloat16)
                q_all.append(jnp.dot(xb, wq_b,
                                     preferred_element_type=jnp.float32))

        q_hs = [q_all[b][:, DH * h:DH * (h + 1)].astype(jnp.bfloat16)
                for b in range(B) for h in range(H_SH)]

        def seg_weights(kbuf, width, col0):
            qi = lax.broadcasted_iota(jnp.int32, (SQ, width), 0)
            kj = lax.broadcasted_iota(jnp.int32, (SQ, width), 1) + col0
            bias = jnp.where(jnp.abs(qi - kj) <= WINDOW, 0.0, -1e9)
            ws, sums = [], []
            for b in range(B):
                k_seg = kbuf[b]
                for h in range(H_SH):
                    s = lax.dot_general(
                        q_hs[b * H_SH + h], k_seg[:, DH * h:DH * (h + 1)],
                        (((1,), (1,)), ((), ())),
                        preferred_element_type=jnp.float32)
                    w = jnp.exp(s + bias)
                    sums.append(jnp.sum(w, axis=1, keepdims=True))
                    ws.append(w.astype(jnp.bfloat16))
            return ws, sums

        with _scope("k0_wait"):
            kv_wait_one(0, 0)
        attn_scope = _scope("attn_segA")
        attn_scope.__enter__()
        wsA, sumsA = seg_weights(kbuf0, SKV_SH, 0)
        attn_scope.__exit__(None, None, None)

        with _scope("k1_wait"):
            kv_wait_one(0, 1)
        attn_scope = _scope("attn_segB")
        attn_scope.__enter__()
        wsB, sumsB = seg_weights(kbuf1, KV1, SKV_SH)
        sums = [a + c for a, c in zip(sumsA, sumsB)]
        attn_scope.__exit__(None, None, None)

        with _scope("v0_wait"):
            kv_wait_one(1, 0)
        attn_scope = _scope("attn_u0")
        attn_scope.__enter__()
        us = []
        for b in range(B):
            v0 = vbuf0[b]
            for h in range(H_SH):
                us.append(jnp.dot(
                    wsA[b * H_SH + h], v0[:, DH * h:DH * (h + 1)],
                    preferred_element_type=jnp.float32))
        attn_scope.__exit__(None, None, None)

        with _scope("v1_wait"):
            kv_wait_one(1, 1)

        def rs_descriptors(b):
            ds = []
            for p in range(1, N_DEV):
                tgt = (my_pos + p) % N_DEV
                ds.append(pltpu.make_async_remote_copy(
                    src_ref=rs_stage.at[b, pl.ds(tgt * QTR, QTR), :],
                    dst_ref=rs_recv.at[p - 1, b],
                    send_sem=rs_send_sem.at[p - 1, b],
                    recv_sem=rs_recv_sem.at[p - 1, b],
                    device_id=(tgt,),
                    device_id_type=pl.DeviceIdType.MESH,
                ))
            return ds

        attn_scope = _scope("attn_ctx")
        attn_scope.__enter__()
        rs_rdmas = []
        for b in range(B):
            v1 = vbuf1[b]
            ctx_cols = []
            for h in range(H_SH):
                i = b * H_SH + h
                u = us[i] + jnp.dot(wsB[i], v1[:, DH * h:DH * (h + 1)],
                                    preferred_element_type=jnp.float32)
                ctx_cols.append(u / sums[i])
            ctx_b = jnp.concatenate(ctx_cols, axis=1).astype(jnp.bfloat16)
            part_b = jnp.dot(ctx_b, wo_b,
                             preferred_element_type=jnp.float32)
            rs_stage[b] = part_b.astype(jnp.bfloat16)
            out_ref[b] = part_b
            if _RDMA:
                ds = rs_descriptors(b)
                for r in ds:
                    r.start()
                rs_rdmas.extend(ds)
        attn_scope.__exit__(None, None, None)

        if _KV:
            with _scope("kv_wait_send"):
                for src in range(2):
                    @pl.when(my_pos == src)
                    def _(src=src):
                        kds, vds = kv_descriptors(src)
                        for d in kds + vds:
                            d.wait_send()

        if _RDMA:
            ag_rdmas = []
            with _scope("reduce_ag"):
                for b in range(B):
                    for r in rs_rdmas[3 * b:3 * (b + 1)]:
                        r.wait_recv()
                    red_b = out_ref[b, pl.ds(my_pos * QTR, QTR), :]
                    for p in range(1, N_DEV):
                        red_b = red_b + rs_recv[p - 1, b].astype(jnp.float32)
                    out_ref[b, pl.ds(my_pos * QTR, QTR), :] = red_b
                    ag_stage[b] = red_b.astype(jnp.bfloat16)
                    for p in range(1, N_DEV):
                        tgt = (my_pos + p) % N_DEV
                        r = pltpu.make_async_remote_copy(
                            src_ref=ag_stage.at[b],
                            dst_ref=ag_recv.at[p - 1, b],
                            send_sem=ag_send_sem.at[p - 1, b],
                            recv_sem=ag_recv_sem.at[p - 1, b],
                            device_id=(tgt,),
                            device_id_type=pl.DeviceIdType.MESH,
                        )
                        r.start()
                        ag_rdmas.append(r)
            with _scope("ag_wait_recv"):
                for r in ag_rdmas:
                    r.wait_recv()
            with _scope("assemble"):
                for p in range(1, N_DEV):
                    src = (my_pos - p) % N_DEV
                    for b in range(B):
                        out_ref[b, pl.ds(src * QTR, QTR), :] = (
                            ag_recv[p - 1, b].astype(jnp.float32))

            with _scope("tail_wait_send"):
                for r in rs_rdmas:
                    r.wait_send()
                for r in ag_rdmas:
                    r.wait_send()

    return pl.pallas_call(
        body,
        out_shape=jax.ShapeDtypeStruct((B, SQ, D_MODEL), jnp.float32),
        in_specs=[pl.BlockSpec(memory_space=pltpu.VMEM)] * 5,
        out_specs=pl.BlockSpec(memory_space=pltpu.VMEM),
        scratch_shapes=[
            pltpu.VMEM((B, SKV_SH, HQ * DH), jnp.bfloat16),
            pltpu.VMEM((B, SKV_SH, HQ * DH), jnp.bfloat16),
            pltpu.VMEM((B, SKV_SH, HD), jnp.bfloat16),
            pltpu.VMEM((B, SKV_SH, HD), jnp.bfloat16),
            pltpu.VMEM((B, KV1, HD), jnp.bfloat16),
            pltpu.VMEM((B, KV1, HD), jnp.bfloat16),
            pltpu.VMEM((B, SQ, D_MODEL), jnp.bfloat16),
            pltpu.VMEM((N_DEV - 1, B, QTR, D_MODEL), jnp.bfloat16),
            pltpu.VMEM((B, QTR, D_MODEL), jnp.bfloat16),
            pltpu.VMEM((N_DEV - 1, B, QTR, D_MODEL), jnp.bfloat16),
            pltpu.SemaphoreType.DMA((N_DEV - 1, 2)),
            pltpu.SemaphoreType.DMA((2, 2)),
            pltpu.SemaphoreType.DMA((N_DEV - 1, B)),
            pltpu.SemaphoreType.DMA((N_DEV - 1, B)),
            pltpu.SemaphoreType.DMA((N_DEV - 1, B)),
            pltpu.SemaphoreType.DMA((N_DEV - 1, B)),
        ],
        compiler_params=(pltpu.CompilerParams(collective_id=0) if _RDMA
                         else pltpu.CompilerParams()),
    )(x, Wq, K_ext, V_ext, Wo)
